# baseline (speedup 1.0000x reference)
import functools
import numpy as np
import jax
import jax.numpy as jnp
from jax import lax
from jax.experimental import pallas as pl
from jax.experimental.pallas import tpu as pltpu
from jax.experimental.pallas import tpu_sc as plsc

VOCAB = 100000
CUTS = (0, 2000, 10000, 50000, 100000)
CUT1, CUT2, CUT3 = 2000, 10000, 50000
H = 768
LPAD = 2048
TT = 256
NTT = LPAD // TT
VT = 1024
NVT = (VOCAB + VT - 1) // VT
MAX_ITEMS = NVT * NTT
LOG2E = 1.4426950408889634
LN2 = 0.6931471805599453

_c_lo = np.array([int(np.searchsorted(CUTS, v * VT, 'right') - 1)
                  for v in range(NVT)], np.int32)
_c_hi = np.array([int(np.searchsorted(CUTS, min((v + 1) * VT, VOCAB) - 1,
                                      'right') - 1)
                  for v in range(NVT)], np.int32)
_pure = (_c_lo == _c_hi).astype(np.int32)
if VOCAB % VT != 0:
    _pure[-1] = 0

_SC_NW = 32
_BPW = LPAD // _SC_NW


def _cluster_of(v):
    return ((v >= CUT1).astype(jnp.int32) + (v >= CUT2).astype(jnp.int32)
            + (v >= CUT3).astype(jnp.int32))


def _schedule(yf):
    n = yf.shape[0]
    cl = _cluster_of(yf)
    oh = (cl[:, None] == jnp.arange(4, dtype=jnp.int32)[None, :])
    pref = jnp.cumsum(oh.astype(jnp.int32), axis=0)
    counts = pref[-1]
    base4 = jnp.concatenate([jnp.zeros((1,), jnp.int32),
                             jnp.cumsum(counts)[:3].astype(jnp.int32)])
    rank = jnp.take_along_axis(pref, cl[:, None], axis=1)[:, 0] - 1
    pos = (base4[cl] + rank).astype(jnp.int32)
    pos_pad = jnp.concatenate(
        [pos, jnp.full((LPAD - n,), LPAD - 1, jnp.int32)])

    offs = jnp.concatenate([jnp.zeros((1,), jnp.int32),
                            jnp.cumsum(counts).astype(jnp.int32)])
    start = offs[_c_lo]
    end = offs[_c_hi + 1]
    tlo = (start // TT).astype(jnp.int32)
    cnt = jnp.where(end > start,
                    (end + TT - 1) // TT - start // TT, 0).astype(jnp.int32)
    csum = jnp.cumsum(cnt)
    base = jnp.concatenate([jnp.zeros((1,), jnp.int32),
                            csum[:-1].astype(jnp.int32)])
    total = csum[-1]
    j = jnp.arange(MAX_ITEMS, dtype=jnp.int32)
    vj = jnp.clip(jnp.searchsorted(base, j, side='right') - 1,
                  0, NVT - 1).astype(jnp.int32)
    tt = jnp.clip(tlo[vj] + (j - base[vj]), 0, NTT - 1).astype(jnp.int32)
    valid = (j < total).astype(jnp.int32)
    pure = jnp.asarray(_pure)[vj]
    cid = jnp.asarray(_c_lo)[vj]
    return pos, pos_pad, tt, vj, valid, pure, cid


def _grouped_body(tt_ref, wt_ref, valid_ref, pure_ref, cid_ref,
                  y_ref, x_ref, w_ref, b_ref, cw_ref, cb_ref,
                  out_ref, s_acc, t_acc, cll):
    j = pl.program_id(0)

    @pl.when(j == 0)
    def _init():
        s_acc[:] = jnp.zeros_like(s_acc)
        t_acc[:] = jnp.zeros_like(t_acc)
        clg = jnp.dot(x_ref[:], cw_ref[:],
                      preferred_element_type=jnp.float32) * LN2 + cb_ref[:]
        m = jnp.max(clg, axis=1, keepdims=True)
        lse = m + jnp.log(jnp.sum(jnp.exp(clg - m), axis=1, keepdims=True))
        ccol = jax.lax.broadcasted_iota(jnp.int32, (1, clg.shape[1]), 1)
        tok_cl = _cluster_of(y_ref[:])
        cll[:] = jnp.sum(jnp.where(ccol == tok_cl, clg - lse, 0.0),
                         axis=1, keepdims=True)

    @pl.when(valid_ref[j] != 0)
    def _item():
        r0 = tt_ref[j] * TT
        wt = wt_ref[j]
        xt = x_ref[pl.ds(r0, TT), :]
        l = jnp.dot(xt, w_ref[:],
                    preferred_element_type=jnp.float32) + b_ref[:]
        col = wt * VT + jax.lax.broadcasted_iota(jnp.int32, (1, VT), 1)
        yt = y_ref[pl.ds(r0, TT), :]
        tok_cl = _cluster_of(yt)
        e = jnp.exp2(l)

        @pl.when(pure_ref[j] == 1)
        def _fast():
            ssum = jnp.sum(e, axis=1, keepdims=True)
            s_acc[pl.ds(r0, TT), :] += jnp.where(
                tok_cl == cid_ref[j], ssum, 0.0)

        @pl.when(pure_ref[j] == 0)
        def _slow():
            col_cl = jnp.where(col < VOCAB, _cluster_of(col), -1)
            s_acc[pl.ds(r0, TT), :] += jnp.sum(
                jnp.where(col_cl == tok_cl, e, 0.0),
                axis=1, keepdims=True)

        t_acc[pl.ds(r0, TT), :] += jnp.sum(
            jnp.where(col == yt, l, 0.0), axis=1, keepdims=True)

    @pl.when(j == MAX_ITEMS - 1)
    def _finish():
        nll = -(cll[:] + LN2 * (t_acc[:] - jnp.log2(s_acc[:])))
        out_ref[:] = jnp.broadcast_to(nll, (LPAD, 128))


def _tc_grouped(x_s, y_s, W, b2, cW, cb, tt, wt, valid, pure, cid):
    grid_spec = pltpu.PrefetchScalarGridSpec(
        num_scalar_prefetch=5,
        grid=(MAX_ITEMS,),
        in_specs=[
            pl.BlockSpec((LPAD, 1), lambda j, *s: (0, 0)),
            pl.BlockSpec((LPAD, H), lambda j, *s: (0, 0)),
            pl.BlockSpec((H, VT), lambda j, t, w, *s: (0, w[j])),
            pl.BlockSpec((1, VT), lambda j, t, w, *s: (0, w[j])),
            pl.BlockSpec(cW.shape, lambda j, *s: (0, 0)),
            pl.BlockSpec(cb.shape, lambda j, *s: (0, 0)),
        ],
        out_specs=pl.BlockSpec((LPAD, 128), lambda j, *s: (0, 0)),
        scratch_shapes=[
            pltpu.VMEM((LPAD, 1), jnp.float32),
            pltpu.VMEM((LPAD, 1), jnp.float32),
            pltpu.VMEM((LPAD, 1), jnp.float32),
        ],
    )
    return pl.pallas_call(
        _grouped_body,
        grid_spec=grid_spec,
        out_shape=jax.ShapeDtypeStruct((LPAD, 128), jnp.float32),
        compiler_params=pltpu.CompilerParams(
            dimension_semantics=("arbitrary",)),
    )(tt, wt, valid, pure, cid, y_s, x_s, W, b2, cW, cb)


def _sc_scatter_x(x_pad, pos_pad):
    mesh = plsc.VectorSubcoreMesh(core_axis_name="c", subcore_axis_name="s")

    @functools.partial(
        pl.kernel, mesh=mesh,
        out_type=jax.ShapeDtypeStruct((LPAD, H), jnp.float32),
        scratch_types=[
            pltpu.VMEM((_BPW,), jnp.int32),
            pltpu.VMEM((_BPW, H), jnp.float32),
            pltpu.SemaphoreType.DMA,
        ],
    )
    def k(x_hbm, idx_hbm, out_hbm, idx_v, rows_v, sem):
        wid = lax.axis_index("s") * 2 + lax.axis_index("c")
        b0 = wid * _BPW
        pltpu.sync_copy(idx_hbm.at[pl.ds(b0, _BPW)], idx_v)
        pltpu.sync_copy(x_hbm.at[pl.ds(b0, _BPW)], rows_v)
        pltpu.async_copy(rows_v, out_hbm.at[idx_v], sem).wait()

    return k(x_pad, pos_pad)


def _sc_gather_out(src, pos_pad):
    mesh = plsc.VectorSubcoreMesh(core_axis_name="c", subcore_axis_name="s")

    @functools.partial(
        pl.kernel, mesh=mesh,
        out_type=jax.ShapeDtypeStruct((LPAD, 128), jnp.float32),
        scratch_types=[
            pltpu.VMEM((_BPW,), jnp.int32),
            pltpu.VMEM((_BPW, 128), jnp.float32),
            pltpu.SemaphoreType.DMA,
        ],
    )
    def k(src_hbm, idx_hbm, out_hbm, idx_v, rows_v, sem):
        wid = lax.axis_index("s") * 2 + lax.axis_index("c")
        b0 = wid * _BPW
        pltpu.sync_copy(idx_hbm.at[pl.ds(b0, _BPW)], idx_v)
        pltpu.async_copy(src_hbm.at[idx_v], rows_v, sem).wait()
        pltpu.sync_copy(rows_v, out_hbm.at[pl.ds(b0, _BPW)])

    return k(src, pos_pad)


def kernel(x, y, cluster_W, cluster_b, W, b):
    x = x[:, :-1]
    bsz, l, h = x.shape
    xf = x.reshape(bsz * l, h)
    yf = y.reshape(-1)
    n = xf.shape[0]
    xp = jnp.pad(xf, ((0, LPAD - n), (0, 0))) * jnp.float32(LOG2E)
    b2 = b * jnp.float32(LOG2E)

    pos, pos_pad, tt, wt, valid, pure, cid = _schedule(yf)
    x_s = _sc_scatter_x(xp, pos_pad)
    y_s = jnp.zeros((LPAD,), jnp.int32).at[pos].set(yf).reshape(LPAD, 1)
    nll_s = _tc_grouped(x_s, y_s, W, b2, cluster_W, cluster_b,
                        tt, wt, valid, pure, cid)
    nll = _sc_gather_out(nll_s, pos_pad)
    return nll[:n, 0]
